# baseline (speedup 1.0000x reference)
import functools

import jax
import jax.numpy as jnp
from jax.experimental import pallas as pl

_KMAX = 0.2
_KMIN = 0.2
_ALPHA = 0.7


def _pos_k(k, n):
    if k <= 0:
        return 0
    elif k < 1:
        return int(round(k * n))
    elif k > n:
        return int(n)
    return int(k)


def _key_fwd(i):
    return jnp.where(i >= 0, i, i ^ jnp.int32(0x7FFFFFFF))


def _count16(mask):
    v = mask.astype(jnp.int16)
    n = v.shape[1]
    while n > 64:
        n //= 2
        v = v[:, :n] + v[:, n:]
    return jnp.sum(v.astype(jnp.int32), axis=1, keepdims=True)


def _descend16(key16, base, kk, num_bits):
    cnt0 = base + _count16(key16 >= jnp.int16(0))
    p = jnp.where(cnt0 >= kk, jnp.int32(0), jnp.int32(-32768))

    def bit_body(t, p):
        bit = jax.lax.shift_left(jnp.int32(1), jnp.int32(num_bits - 2) - t)
        c = p + bit
        c16 = c.astype(jnp.int16)
        cnt = base + _count16(key16 >= c16)
        return jnp.where(cnt >= kk, c, p)

    return jax.lax.fori_loop(0, num_bits - 1, bit_body, p)


def _body(k, alpha, x_ref, o_ref):
    x = x_ref[...]
    i = jax.lax.bitcast_convert_type(x, jnp.int32)
    ikey = _key_fwd(i)
    hkey = jax.lax.shift_right_arithmetic(ikey, jnp.int32(16)).astype(
        jnp.int16)
    gkey = ~hkey
    kk = jnp.int32(k)
    r = x.shape[0]
    zero16 = jnp.zeros((r, 1), jnp.int32)

    h_hi = _descend16(hkey, zero16, kk, 16)
    h_lo = _descend16(gkey, zero16, kk, 16)

    h_hi16 = h_hi.astype(jnp.int16)
    h_lo16 = h_lo.astype(jnp.int16)
    low = ikey & jnp.int32(0xFFFF)
    pmax = jnp.int16(32767)
    pmin = jnp.int16(-32768)

    lk_hi = jnp.where(
        hkey > h_hi16, pmax,
        jnp.where(hkey == h_hi16,
                  (low - jnp.int32(32768)).astype(jnp.int16), pmin))

    lk_lo = jnp.where(
        gkey > h_lo16, pmax,
        jnp.where(gkey == h_lo16,
                  (jnp.int32(32767) - low).astype(jnp.int16), pmin))

    b_hi = _descend16(lk_hi, zero16, kk, 16)
    b_lo = _descend16(lk_lo, zero16, kk, 16)

    key_hi = jax.lax.shift_left(h_hi, jnp.int32(16)) | (
        b_hi + jnp.int32(32768))
    key_lo = jax.lax.shift_left(h_lo, jnp.int32(16)) | (
        b_lo + jnp.int32(32768))

    def key_to_f32(kv):
        return jax.lax.bitcast_convert_type(_key_fwd(kv), jnp.float32)

    t_hi = key_to_f32(key_hi)
    t_lo = key_to_f32(~key_lo)
    kf = jnp.float32(k)
    s_top = kf * t_hi[:, 0] + jnp.sum(jnp.maximum(x - t_hi, 0.0), axis=1)
    s_bot = kf * t_lo[:, 0] - jnp.sum(jnp.maximum(t_lo - x, 0.0), axis=1)
    out = (s_top + jnp.float32(alpha) * s_bot) * jnp.float32(1.0 / (2.0 * k))
    o_ref[...] = out.reshape(r, 1)


def kernel(input):
    b, c, h, w = input.shape
    n = h * w
    kmax = _pos_k(_KMAX, n)
    num_rows = b * c
    flat = input.reshape(num_rows, n)
    r = 512
    assert num_rows % r == 0
    out = pl.pallas_call(
        functools.partial(_body, kmax, _ALPHA),
        grid=(num_rows // r,),
        in_specs=[pl.BlockSpec((r, n), lambda i: (i, 0))],
        out_specs=pl.BlockSpec((r, 1), lambda i: (i, 0)),
        out_shape=jax.ShapeDtypeStruct((num_rows, 1), jnp.float32),
    )(flat)
    return out.reshape(b, c)


# speedup vs baseline: 1.2872x; 1.2872x over previous
import functools

import jax
import jax.numpy as jnp
from jax.experimental import pallas as pl

_KMAX = 0.2
_KMIN = 0.2
_ALPHA = 0.7
_INT_MIN = -2147483648


def _pos_k(k, n):
    if k <= 0:
        return 0
    elif k < 1:
        return int(round(k * n))
    elif k > n:
        return int(n)
    return int(k)


def _key_fwd(i):
    return jnp.where(i >= 0, i, i ^ jnp.int32(0x7FFFFFFF))


def _body(k, alpha, x_ref, o_ref):
    x = x_ref[...]
    xt = x.T
    ikey = _key_fwd(jax.lax.bitcast_convert_type(xt, jnp.int32))
    kk = jnp.int32(k)
    r = x.shape[0]
    imin = jnp.int32(_INT_MIN)

    def counts(c_hi, c_lo):
        v = (jnp.where(ikey >= c_hi, 1, 0)
             + jnp.where(ikey <= ~c_lo, 65536, 0))
        s = jnp.sum(v, axis=0, keepdims=True)
        return s & jnp.int32(0xFFFF), jax.lax.shift_right_logical(
            s, jnp.int32(16))

    zero = jnp.zeros((1, r), jnp.int32)
    cnt0_hi, cnt0_lo = counts(zero, zero)
    init_hi = jnp.where(cnt0_hi >= kk, jnp.int32(0), imin)
    init_lo = jnp.where(cnt0_lo >= kk, jnp.int32(0), imin)

    def bit_body(t, carry):
        p_hi, p_lo = carry
        bit = jax.lax.shift_left(jnp.int32(1), jnp.int32(30) - t)
        c_hi = p_hi + bit
        c_lo = p_lo + bit
        cnt_hi, cnt_lo = counts(c_hi, c_lo)
        p_hi = jnp.where(cnt_hi >= kk, c_hi, p_hi)
        p_lo = jnp.where(cnt_lo >= kk, c_lo, p_lo)
        return p_hi, p_lo

    p_hi, p_lo = jax.lax.fori_loop(0, 31, bit_body, (init_hi, init_lo))

    def key_to_f32(kv):
        return jax.lax.bitcast_convert_type(_key_fwd(kv), jnp.float32)

    t_hi = key_to_f32(p_hi)
    t_lo = key_to_f32(~p_lo)
    kf = jnp.float32(k)
    s_top = kf * t_hi + jnp.sum(jnp.maximum(xt - t_hi, 0.0), axis=0,
                                keepdims=True)
    s_bot = kf * t_lo - jnp.sum(jnp.maximum(t_lo - xt, 0.0), axis=0,
                                keepdims=True)
    out = (s_top + jnp.float32(alpha) * s_bot) * jnp.float32(1.0 / (2.0 * k))
    o_ref[...] = out.reshape(1, 1, r)


def kernel(input):
    b, c, h, w = input.shape
    n = h * w
    kmax = _pos_k(_KMAX, n)
    num_rows = b * c
    flat = input.reshape(num_rows, n)
    r = 256
    assert num_rows % r == 0
    out = pl.pallas_call(
        functools.partial(_body, kmax, _ALPHA),
        grid=(num_rows // r,),
        in_specs=[pl.BlockSpec((r, n), lambda i: (i, 0))],
        out_specs=pl.BlockSpec((1, 1, r), lambda i: (i, 0, 0)),
        out_shape=jax.ShapeDtypeStruct((num_rows // r, 1, r), jnp.float32),
    )(flat)
    return out.reshape(b, c)
